# initial kernel scaffold (unmeasured)
import jax
import jax.numpy as jnp
from jax import lax
from jax.experimental import pallas as pl
from jax.experimental.pallas import tpu as pltpu

N_DEV = 4


def kernel(x, w_mat):
    m, k_local = x.shape
    _, n = w_mat.shape
    m_chunk = m // N_DEV

    def body(x_ref, w_ref, out_ref, send_ref, recv_ref, send_sem, recv_sems):
        my = lax.axis_index("i")
        left = (my + N_DEV - 1) % N_DEV
        right = (my + 1) % N_DEV

        barrier_sem = pltpu.get_barrier_semaphore()
        for nbr in [left, right]:
            pl.semaphore_signal(
                barrier_sem, inc=1,
                device_id=(nbr,), device_id_type=pl.DeviceIdType.MESH,
            )
        pl.semaphore_wait(barrier_sem, 2)

        def partial_chunk(c):
            return jnp.dot(
                x_ref[pl.ds(c * m_chunk, m_chunk), :], w_ref[:, :],
                preferred_element_type=jnp.float32,
            )

        send_ref[:, :] = partial_chunk((my + N_DEV - 1) % N_DEV)

        for h in range(N_DEV - 1):
            rdma = pltpu.make_async_remote_copy(
                src_ref=send_ref,
                dst_ref=recv_ref.at[h],
                send_sem=send_sem,
                recv_sem=recv_sems.at[h],
                device_id=(right,),
                device_id_type=pl.DeviceIdType.MESH,
            )
            rdma.start()
            rdma.wait()

            c = (my + 2 * N_DEV - h - 2) % N_DEV
            if h < N_DEV - 2:
                send_ref[:, :] = recv_ref[h] + partial_chunk(c)
            else:
                out_ref[:, :] = jnp.maximum(recv_ref[h] + partial_chunk(c), 0.0)

    return pl.pallas_call(
        body,
        out_shape=jax.ShapeDtypeStruct((m_chunk, n), jnp.float32),
        in_specs=[
            pl.BlockSpec(memory_space=pltpu.VMEM),
            pl.BlockSpec(memory_space=pltpu.VMEM),
        ],
        out_specs=pl.BlockSpec(memory_space=pltpu.VMEM),
        scratch_shapes=[
            pltpu.VMEM((m_chunk, n), jnp.float32),
            pltpu.VMEM((N_DEV - 1, m_chunk, n), jnp.float32),
            pltpu.SemaphoreType.DMA,
            pltpu.SemaphoreType.DMA((N_DEV - 1,)),
        ],
        compiler_params=pltpu.CompilerParams(collective_id=0),
    )(x, w_mat)


# baseline (device time: 321044 ns/iter reference)
import jax
import jax.numpy as jnp
from jax import lax
from jax.experimental import pallas as pl
from jax.experimental.pallas import tpu as pltpu

N_DEV = 4


def kernel(x, w_mat):
    m, k_local = x.shape
    _, n = w_mat.shape
    m_chunk = m // N_DEV

    def body(x_ref, w_ref, out_ref, send_ref, recv_ref, xchunk_ref,
             send_sem, recv_sems, copy_sem):
        my = lax.axis_index("i")
        left = (my + N_DEV - 1) % N_DEV
        right = (my + 1) % N_DEV

        barrier_sem = pltpu.get_barrier_semaphore()
        for nbr in [left, right]:
            pl.semaphore_signal(
                barrier_sem, inc=1,
                device_id=(nbr,), device_id_type=pl.DeviceIdType.MESH,
            )
        pl.semaphore_wait(barrier_sem, 2)

        def partial_chunk(c):
            cp = pltpu.make_async_copy(
                x_ref.at[pl.ds(c * m_chunk, m_chunk), :], xchunk_ref, copy_sem,
            )
            cp.start()
            cp.wait()
            return jnp.dot(
                xchunk_ref[:, :], w_ref[:, :],
                preferred_element_type=jnp.float32,
            )

        send_ref[:, :] = partial_chunk((my + N_DEV - 1) % N_DEV)

        for h in range(N_DEV - 1):
            rdma = pltpu.make_async_remote_copy(
                src_ref=send_ref,
                dst_ref=recv_ref.at[h],
                send_sem=send_sem,
                recv_sem=recv_sems.at[h],
                device_id=(right,),
                device_id_type=pl.DeviceIdType.MESH,
            )
            rdma.start()
            rdma.wait()

            c = (my + 2 * N_DEV - h - 2) % N_DEV
            if h < N_DEV - 2:
                send_ref[:, :] = recv_ref[h] + partial_chunk(c)
            else:
                out_ref[:, :] = jnp.maximum(recv_ref[h] + partial_chunk(c), 0.0)

    return pl.pallas_call(
        body,
        out_shape=jax.ShapeDtypeStruct((m_chunk, n), jnp.float32),
        in_specs=[
            pl.BlockSpec(memory_space=pltpu.MemorySpace.HBM),
            pl.BlockSpec(memory_space=pltpu.VMEM),
        ],
        out_specs=pl.BlockSpec(memory_space=pltpu.VMEM),
        scratch_shapes=[
            pltpu.VMEM((m_chunk, n), jnp.float32),
            pltpu.VMEM((N_DEV - 1, m_chunk, n), jnp.float32),
            pltpu.VMEM((m_chunk, k_local), jnp.float32),
            pltpu.SemaphoreType.DMA,
            pltpu.SemaphoreType.DMA((N_DEV - 1,)),
            pltpu.SemaphoreType.DMA,
        ],
        compiler_params=pltpu.CompilerParams(
            collective_id=0,
            vmem_limit_bytes=128 * 1024 * 1024,
        ),
    )(x, w_mat)


# device time: 166180 ns/iter; 1.9319x vs baseline; 1.9319x over previous
import jax
import jax.numpy as jnp
from jax import lax
from jax.experimental import pallas as pl
from jax.experimental.pallas import tpu as pltpu

N_DEV = 4


def kernel(x, w_mat):
    m, k_local = x.shape
    _, n = w_mat.shape
    m_chunk = m // N_DEV
    nh = n // 2

    f32 = jnp.float32

    def body(x_ref, w_ref, out_ref, scw_ref, sccw_ref, rcw_ref, rccw_ref,
             xch_ref, send_sems, rcw_sems, rccw_sems, copy_sems):
        my = lax.axis_index("i")
        left = (my + N_DEV - 1) % N_DEV
        right = (my + 1) % N_DEV

        barrier_sem = pltpu.get_barrier_semaphore()
        for nbr in [left, right]:
            pl.semaphore_signal(
                barrier_sem, inc=1,
                device_id=(nbr,), device_id_type=pl.DeviceIdType.MESH,
            )
        pl.semaphore_wait(barrier_sem, 2)

        def load_x(c, slot, sem_slot):
            cp = pltpu.make_async_copy(
                x_ref.at[pl.ds(c * m_chunk, m_chunk), :],
                xch_ref.at[slot],
                copy_sems.at[sem_slot],
            )
            cp.start()
            return cp

        def dot(xslot, col_lo, col_hi):
            return jnp.dot(
                xch_ref[xslot, :, :], w_ref[:, col_lo:col_hi],
                preferred_element_type=f32,
            )

        def send_cw(h):
            rdma = pltpu.make_async_remote_copy(
                src_ref=scw_ref, dst_ref=rcw_ref.at[h],
                send_sem=send_sems.at[0], recv_sem=rcw_sems.at[h],
                device_id=(right,), device_id_type=pl.DeviceIdType.MESH,
            )
            rdma.start()
            return rdma

        def send_ccw(h):
            rdma = pltpu.make_async_remote_copy(
                src_ref=sccw_ref, dst_ref=rccw_ref.at[h],
                send_sem=send_sems.at[1], recv_sem=rccw_sems.at[h],
                device_id=(left,), device_id_type=pl.DeviceIdType.MESH,
            )
            rdma.start()
            return rdma

        cp0 = load_x((my + N_DEV - 1) % N_DEV, 0, 0)
        cp1 = load_x((my + 1) % N_DEV, 1, 1)

        cp0.wait()
        scw_ref[:, :] = dot(0, 0, nh)
        cw0 = send_cw(0)
        cp1.wait()
        sccw_ref[:, :] = dot(1, nh, n)
        ccw0 = send_ccw(0)

        cp2 = load_x((my + 2) % N_DEV, 2, 0)
        cp2.wait()
        out_ref[:, :] = jnp.dot(
            xch_ref[2, :, :], w_ref[:, :], preferred_element_type=f32,
        )

        cw0.wait()
        ccw0.wait()

        scw_ref[:, :] = rcw_ref[0] + out_ref[:, 0:nh]
        cw1 = send_cw(1)
        sccw_ref[:, :] = rccw_ref[0] + out_ref[:, nh:n]
        ccw1 = send_ccw(1)

        out_ref[:, 0:nh] = dot(1, 0, nh)
        out_ref[:, nh:n] = dot(0, nh, n)

        cw1.wait()
        ccw1.wait()

        scw_ref[:, :] = rcw_ref[1] + out_ref[:, 0:nh]
        cw2 = send_cw(2)
        sccw_ref[:, :] = rccw_ref[1] + out_ref[:, nh:n]
        ccw2 = send_ccw(2)

        cp3 = load_x(my, 2, 0)
        cp3.wait()
        out_ref[:, :] = jnp.dot(
            xch_ref[2, :, :], w_ref[:, :], preferred_element_type=f32,
        )

        cw2.wait()
        ccw2.wait()

        out_ref[:, 0:nh] = jnp.maximum(out_ref[:, 0:nh] + rcw_ref[2], 0.0)
        out_ref[:, nh:n] = jnp.maximum(out_ref[:, nh:n] + rccw_ref[2], 0.0)

    return pl.pallas_call(
        body,
        out_shape=jax.ShapeDtypeStruct((m_chunk, n), f32),
        in_specs=[
            pl.BlockSpec(memory_space=pltpu.MemorySpace.HBM),
            pl.BlockSpec(memory_space=pltpu.VMEM),
        ],
        out_specs=pl.BlockSpec(memory_space=pltpu.VMEM),
        scratch_shapes=[
            pltpu.VMEM((m_chunk, nh), f32),
            pltpu.VMEM((m_chunk, nh), f32),
            pltpu.VMEM((N_DEV - 1, m_chunk, nh), f32),
            pltpu.VMEM((N_DEV - 1, m_chunk, nh), f32),
            pltpu.VMEM((3, m_chunk, k_local), f32),
            pltpu.SemaphoreType.DMA((2,)),
            pltpu.SemaphoreType.DMA((N_DEV - 1,)),
            pltpu.SemaphoreType.DMA((N_DEV - 1,)),
            pltpu.SemaphoreType.DMA((2,)),
        ],
        compiler_params=pltpu.CompilerParams(
            collective_id=0,
            vmem_limit_bytes=128 * 1024 * 1024,
        ),
    )(x, w_mat)


# device time: 158291 ns/iter; 2.0282x vs baseline; 1.0498x over previous
import jax
import jax.numpy as jnp
from jax import lax
from jax.experimental import pallas as pl
from jax.experimental.pallas import tpu as pltpu

N_DEV = 4
SEG = 4


def kernel(x, w_mat):
    m, k_local = x.shape
    _, n = w_mat.shape
    m_chunk = m // N_DEV
    nh = n // 2
    seg = m_chunk // SEG

    f32 = jnp.float32

    def body(x_ref, w_ref, out_ref, scw_ref, sccw_ref, rcw_ref, rccw_ref,
             xch_ref, scw_sems, sccw_sems, rcw_sems, rccw_sems, copy_sems):
        my = lax.axis_index("i")
        left = (my + N_DEV - 1) % N_DEV
        right = (my + 1) % N_DEV

        barrier_sem = pltpu.get_barrier_semaphore()
        for nbr in [left, right]:
            pl.semaphore_signal(
                barrier_sem, inc=1,
                device_id=(nbr,), device_id_type=pl.DeviceIdType.MESH,
            )
        pl.semaphore_wait(barrier_sem, 2)

        def load_x(c, slot, sem_slot):
            cp = pltpu.make_async_copy(
                x_ref.at[pl.ds(c * m_chunk, m_chunk), :],
                xch_ref.at[slot],
                copy_sems.at[sem_slot],
            )
            cp.start()
            return cp

        def mk_cw(h, s):
            r0 = s * seg
            return pltpu.make_async_remote_copy(
                src_ref=scw_ref.at[pl.ds(r0, seg), :],
                dst_ref=rcw_ref.at[h, pl.ds(r0, seg), :],
                send_sem=scw_sems.at[s],
                recv_sem=rcw_sems.at[h, s],
                device_id=(right,), device_id_type=pl.DeviceIdType.MESH,
            )

        def mk_ccw(h, s):
            r0 = s * seg
            return pltpu.make_async_remote_copy(
                src_ref=sccw_ref.at[pl.ds(r0, seg), :],
                dst_ref=rccw_ref.at[h, pl.ds(r0, seg), :],
                send_sem=sccw_sems.at[s],
                recv_sem=rccw_sems.at[h, s],
                device_id=(left,), device_id_type=pl.DeviceIdType.MESH,
            )

        cp0 = load_x((my + N_DEV - 1) % N_DEV, 0, 0)
        cp1 = load_x((my + 1) % N_DEV, 1, 1)
        cp2 = load_x((my + 2) % N_DEV, 2, 2)

        cp0.wait()
        cp1.wait()
        prev_cw = [None] * SEG
        prev_ccw = [None] * SEG
        for s in range(SEG):
            r0 = s * seg
            scw_ref[r0:r0 + seg, :] = jnp.dot(
                xch_ref[0, r0:r0 + seg, :], w_ref[:, 0:nh],
                preferred_element_type=f32,
            )
            prev_cw[s] = mk_cw(0, s)
            prev_cw[s].start()
            sccw_ref[r0:r0 + seg, :] = jnp.dot(
                xch_ref[1, r0:r0 + seg, :], w_ref[:, nh:n],
                preferred_element_type=f32,
            )
            prev_ccw[s] = mk_ccw(0, s)
            prev_ccw[s].start()

        cp2.wait()
        out_ref[:, :] = jnp.dot(
            xch_ref[2, :, :], w_ref[:, :], preferred_element_type=f32,
        )

        for s in range(SEG):
            r0 = s * seg
            prev_cw[s].wait()
            scw_ref[r0:r0 + seg, :] = (
                rcw_ref[0, r0:r0 + seg, :] + out_ref[r0:r0 + seg, 0:nh]
            )
            prev_cw[s] = mk_cw(1, s)
            prev_cw[s].start()
            prev_ccw[s].wait()
            sccw_ref[r0:r0 + seg, :] = (
                rccw_ref[0, r0:r0 + seg, :] + out_ref[r0:r0 + seg, nh:n]
            )
            prev_ccw[s] = mk_ccw(1, s)
            prev_ccw[s].start()

        cp3 = load_x(my, 2, 2)
        out_ref[:, 0:nh] = jnp.dot(
            xch_ref[1, :, :], w_ref[:, 0:nh], preferred_element_type=f32,
        )
        out_ref[:, nh:n] = jnp.dot(
            xch_ref[0, :, :], w_ref[:, nh:n], preferred_element_type=f32,
        )

        for s in range(SEG):
            r0 = s * seg
            prev_cw[s].wait()
            scw_ref[r0:r0 + seg, :] = (
                rcw_ref[1, r0:r0 + seg, :] + out_ref[r0:r0 + seg, 0:nh]
            )
            prev_cw[s] = mk_cw(2, s)
            prev_cw[s].start()
            prev_ccw[s].wait()
            sccw_ref[r0:r0 + seg, :] = (
                rccw_ref[1, r0:r0 + seg, :] + out_ref[r0:r0 + seg, nh:n]
            )
            prev_ccw[s] = mk_ccw(2, s)
            prev_ccw[s].start()

        cp3.wait()
        out_ref[:, :] = jnp.dot(
            xch_ref[2, :, :], w_ref[:, :], preferred_element_type=f32,
        )

        for s in range(SEG):
            r0 = s * seg
            prev_cw[s].wait()
            out_ref[r0:r0 + seg, 0:nh] = jnp.maximum(
                out_ref[r0:r0 + seg, 0:nh] + rcw_ref[2, r0:r0 + seg, :], 0.0,
            )
            prev_ccw[s].wait()
            out_ref[r0:r0 + seg, nh:n] = jnp.maximum(
                out_ref[r0:r0 + seg, nh:n] + rccw_ref[2, r0:r0 + seg, :], 0.0,
            )

    return pl.pallas_call(
        body,
        out_shape=jax.ShapeDtypeStruct((m_chunk, n), f32),
        in_specs=[
            pl.BlockSpec(memory_space=pltpu.MemorySpace.HBM),
            pl.BlockSpec(memory_space=pltpu.VMEM),
        ],
        out_specs=pl.BlockSpec(memory_space=pltpu.VMEM),
        scratch_shapes=[
            pltpu.VMEM((m_chunk, nh), f32),
            pltpu.VMEM((m_chunk, nh), f32),
            pltpu.VMEM((N_DEV - 1, m_chunk, nh), f32),
            pltpu.VMEM((N_DEV - 1, m_chunk, nh), f32),
            pltpu.VMEM((3, m_chunk, k_local), f32),
            pltpu.SemaphoreType.DMA((SEG,)),
            pltpu.SemaphoreType.DMA((SEG,)),
            pltpu.SemaphoreType.DMA((N_DEV - 1, SEG)),
            pltpu.SemaphoreType.DMA((N_DEV - 1, SEG)),
            pltpu.SemaphoreType.DMA((3,)),
        ],
        compiler_params=pltpu.CompilerParams(
            collective_id=0,
            vmem_limit_bytes=128 * 1024 * 1024,
        ),
    )(x, w_mat)


# device time: 158066 ns/iter; 2.0311x vs baseline; 1.0014x over previous
import jax
import jax.numpy as jnp
from jax import lax
from jax.experimental import pallas as pl
from jax.experimental.pallas import tpu as pltpu

N_DEV = 4
SEG = 8


def kernel(x, w_mat):
    m, k_local = x.shape
    _, n = w_mat.shape
    m_chunk = m // N_DEV
    nh = n // 2
    seg = m_chunk // SEG

    f32 = jnp.float32

    def body(x_ref, w_ref, out_ref, scw_ref, sccw_ref, rcw_ref, rccw_ref,
             xch_ref, scw_sems, sccw_sems, rcw_sems, rccw_sems, copy_sems):
        my = lax.axis_index("i")
        left = (my + N_DEV - 1) % N_DEV
        right = (my + 1) % N_DEV

        barrier_sem = pltpu.get_barrier_semaphore()
        for nbr in [left, right]:
            pl.semaphore_signal(
                barrier_sem, inc=1,
                device_id=(nbr,), device_id_type=pl.DeviceIdType.MESH,
            )
        pl.semaphore_wait(barrier_sem, 2)

        def load_x(c, slot, sem_slot):
            cp = pltpu.make_async_copy(
                x_ref.at[pl.ds(c * m_chunk, m_chunk), :],
                xch_ref.at[slot],
                copy_sems.at[sem_slot],
            )
            cp.start()
            return cp

        def mk_cw(h, s):
            r0 = s * seg
            return pltpu.make_async_remote_copy(
                src_ref=scw_ref.at[pl.ds(r0, seg), :],
                dst_ref=rcw_ref.at[h, pl.ds(r0, seg), :],
                send_sem=scw_sems.at[s],
                recv_sem=rcw_sems.at[h, s],
                device_id=(right,), device_id_type=pl.DeviceIdType.MESH,
            )

        def mk_ccw(h, s):
            r0 = s * seg
            return pltpu.make_async_remote_copy(
                src_ref=sccw_ref.at[pl.ds(r0, seg), :],
                dst_ref=rccw_ref.at[h, pl.ds(r0, seg), :],
                send_sem=sccw_sems.at[s],
                recv_sem=rccw_sems.at[h, s],
                device_id=(left,), device_id_type=pl.DeviceIdType.MESH,
            )

        cp0 = load_x((my + N_DEV - 1) % N_DEV, 0, 0)
        cp1 = load_x((my + 1) % N_DEV, 1, 1)
        cp2 = load_x((my + 2) % N_DEV, 2, 2)

        cp0.wait()
        cp1.wait()
        prev_cw = [None] * SEG
        prev_ccw = [None] * SEG
        for s in range(SEG):
            r0 = s * seg
            scw_ref[r0:r0 + seg, :] = jnp.dot(
                xch_ref[0, r0:r0 + seg, :], w_ref[:, 0:nh],
                preferred_element_type=f32,
            )
            prev_cw[s] = mk_cw(0, s)
            prev_cw[s].start()
            sccw_ref[r0:r0 + seg, :] = jnp.dot(
                xch_ref[1, r0:r0 + seg, :], w_ref[:, nh:n],
                preferred_element_type=f32,
            )
            prev_ccw[s] = mk_ccw(0, s)
            prev_ccw[s].start()

        cp2.wait()
        out_ref[:, :] = jnp.dot(
            xch_ref[2, :, :], w_ref[:, :], preferred_element_type=f32,
        )

        for s in range(SEG):
            r0 = s * seg
            prev_cw[s].wait()
            scw_ref[r0:r0 + seg, :] = (
                rcw_ref[0, r0:r0 + seg, :] + out_ref[r0:r0 + seg, 0:nh]
            )
            prev_cw[s] = mk_cw(1, s)
            prev_cw[s].start()
            prev_ccw[s].wait()
            sccw_ref[r0:r0 + seg, :] = (
                rccw_ref[0, r0:r0 + seg, :] + out_ref[r0:r0 + seg, nh:n]
            )
            prev_ccw[s] = mk_ccw(1, s)
            prev_ccw[s].start()

        cp3 = load_x(my, 2, 2)
        out_ref[:, 0:nh] = jnp.dot(
            xch_ref[1, :, :], w_ref[:, 0:nh], preferred_element_type=f32,
        )
        out_ref[:, nh:n] = jnp.dot(
            xch_ref[0, :, :], w_ref[:, nh:n], preferred_element_type=f32,
        )

        for s in range(SEG):
            r0 = s * seg
            prev_cw[s].wait()
            scw_ref[r0:r0 + seg, :] = (
                rcw_ref[1, r0:r0 + seg, :] + out_ref[r0:r0 + seg, 0:nh]
            )
            prev_cw[s] = mk_cw(2, s)
            prev_cw[s].start()
            prev_ccw[s].wait()
            sccw_ref[r0:r0 + seg, :] = (
                rccw_ref[1, r0:r0 + seg, :] + out_ref[r0:r0 + seg, nh:n]
            )
            prev_ccw[s] = mk_ccw(2, s)
            prev_ccw[s].start()

        cp3.wait()
        out_ref[:, :] = jnp.dot(
            xch_ref[2, :, :], w_ref[:, :], preferred_element_type=f32,
        )

        for s in range(SEG):
            r0 = s * seg
            prev_cw[s].wait()
            out_ref[r0:r0 + seg, 0:nh] = jnp.maximum(
                out_ref[r0:r0 + seg, 0:nh] + rcw_ref[2, r0:r0 + seg, :], 0.0,
            )
            prev_ccw[s].wait()
            out_ref[r0:r0 + seg, nh:n] = jnp.maximum(
                out_ref[r0:r0 + seg, nh:n] + rccw_ref[2, r0:r0 + seg, :], 0.0,
            )

    return pl.pallas_call(
        body,
        out_shape=jax.ShapeDtypeStruct((m_chunk, n), f32),
        in_specs=[
            pl.BlockSpec(memory_space=pltpu.MemorySpace.HBM),
            pl.BlockSpec(memory_space=pltpu.VMEM),
        ],
        out_specs=pl.BlockSpec(memory_space=pltpu.VMEM),
        scratch_shapes=[
            pltpu.VMEM((m_chunk, nh), f32),
            pltpu.VMEM((m_chunk, nh), f32),
            pltpu.VMEM((N_DEV - 1, m_chunk, nh), f32),
            pltpu.VMEM((N_DEV - 1, m_chunk, nh), f32),
            pltpu.VMEM((3, m_chunk, k_local), f32),
            pltpu.SemaphoreType.DMA((SEG,)),
            pltpu.SemaphoreType.DMA((SEG,)),
            pltpu.SemaphoreType.DMA((N_DEV - 1, SEG)),
            pltpu.SemaphoreType.DMA((N_DEV - 1, SEG)),
            pltpu.SemaphoreType.DMA((3,)),
        ],
        compiler_params=pltpu.CompilerParams(
            collective_id=0,
            vmem_limit_bytes=128 * 1024 * 1024,
        ),
    )(x, w_mat)


# device time: 156294 ns/iter; 2.0541x vs baseline; 1.0113x over previous
import jax
import jax.numpy as jnp
from jax import lax
from jax.experimental import pallas as pl
from jax.experimental.pallas import tpu as pltpu

N_DEV = 4
SEG = 4


def kernel(x, w_mat):
    m, k_local = x.shape
    _, n = w_mat.shape
    m_chunk = m // N_DEV
    nh = n // 2
    seg = m_chunk // SEG

    f32 = jnp.float32

    def body(x_ref, w_ref, out_ref, scw_ref, sccw_ref, rcw_ref, rccw_ref,
             xch_ref, stage_ref, scw_sems, sccw_sems, rcw_sems, rccw_sems,
             copy_sems, out_sems):
        my = lax.axis_index("i")
        left = (my + N_DEV - 1) % N_DEV
        right = (my + 1) % N_DEV

        barrier_sem = pltpu.get_barrier_semaphore()
        for nbr in [left, right]:
            pl.semaphore_signal(
                barrier_sem, inc=1,
                device_id=(nbr,), device_id_type=pl.DeviceIdType.MESH,
            )
        pl.semaphore_wait(barrier_sem, 2)

        def load_x(c, slot, sem_slot):
            cp = pltpu.make_async_copy(
                x_ref.at[pl.ds(c * m_chunk, m_chunk), :],
                xch_ref.at[slot],
                copy_sems.at[sem_slot],
            )
            cp.start()
            return cp

        def mk_cw(h, s):
            r0 = s * seg
            return pltpu.make_async_remote_copy(
                src_ref=scw_ref.at[pl.ds(r0, seg), :],
                dst_ref=rcw_ref.at[h, pl.ds(r0, seg), :],
                send_sem=scw_sems.at[s],
                recv_sem=rcw_sems.at[h, s],
                device_id=(right,), device_id_type=pl.DeviceIdType.MESH,
            )

        def mk_ccw(h, s):
            r0 = s * seg
            return pltpu.make_async_remote_copy(
                src_ref=sccw_ref.at[pl.ds(r0, seg), :],
                dst_ref=rccw_ref.at[h, pl.ds(r0, seg), :],
                send_sem=sccw_sems.at[s],
                recv_sem=rccw_sems.at[h, s],
                device_id=(left,), device_id_type=pl.DeviceIdType.MESH,
            )

        cp0 = load_x((my + N_DEV - 1) % N_DEV, 0, 0)
        cp1 = load_x((my + 1) % N_DEV, 1, 1)
        cp2 = load_x((my + 2) % N_DEV, 2, 2)

        cp0.wait()
        cp1.wait()
        prev_cw = [None] * SEG
        prev_ccw = [None] * SEG
        for s in range(SEG):
            r0 = s * seg
            scw_ref[r0:r0 + seg, :] = jnp.dot(
                xch_ref[0, r0:r0 + seg, :], w_ref[:, 0:nh],
                preferred_element_type=f32,
            )
            prev_cw[s] = mk_cw(0, s)
            prev_cw[s].start()
            sccw_ref[r0:r0 + seg, :] = jnp.dot(
                xch_ref[1, r0:r0 + seg, :], w_ref[:, nh:n],
                preferred_element_type=f32,
            )
            prev_ccw[s] = mk_ccw(0, s)
            prev_ccw[s].start()

        cp2.wait()
        stage_ref[:, :] = jnp.dot(
            xch_ref[2, :, :], w_ref[:, :], preferred_element_type=f32,
        )

        for s in range(SEG):
            r0 = s * seg
            prev_cw[s].wait()
            scw_ref[r0:r0 + seg, :] = (
                rcw_ref[0, r0:r0 + seg, :] + stage_ref[r0:r0 + seg, 0:nh]
            )
            prev_cw[s] = mk_cw(1, s)
            prev_cw[s].start()
            prev_ccw[s].wait()
            sccw_ref[r0:r0 + seg, :] = (
                rccw_ref[0, r0:r0 + seg, :] + stage_ref[r0:r0 + seg, nh:n]
            )
            prev_ccw[s] = mk_ccw(1, s)
            prev_ccw[s].start()

        cp3 = load_x(my, 2, 2)
        stage_ref[:, 0:nh] = jnp.dot(
            xch_ref[1, :, :], w_ref[:, 0:nh], preferred_element_type=f32,
        )
        stage_ref[:, nh:n] = jnp.dot(
            xch_ref[0, :, :], w_ref[:, nh:n], preferred_element_type=f32,
        )

        for s in range(SEG):
            r0 = s * seg
            prev_cw[s].wait()
            scw_ref[r0:r0 + seg, :] = (
                rcw_ref[1, r0:r0 + seg, :] + stage_ref[r0:r0 + seg, 0:nh]
            )
            prev_cw[s] = mk_cw(2, s)
            prev_cw[s].start()
            prev_ccw[s].wait()
            sccw_ref[r0:r0 + seg, :] = (
                rccw_ref[1, r0:r0 + seg, :] + stage_ref[r0:r0 + seg, nh:n]
            )
            prev_ccw[s] = mk_ccw(2, s)
            prev_ccw[s].start()

        cp3.wait()
        stage_ref[:, :] = jnp.dot(
            xch_ref[2, :, :], w_ref[:, :], preferred_element_type=f32,
        )

        out_cps = []
        for s in range(SEG):
            r0 = s * seg
            prev_cw[s].wait()
            scw_ref[r0:r0 + seg, :] = jnp.maximum(
                stage_ref[r0:r0 + seg, 0:nh] + rcw_ref[2, r0:r0 + seg, :], 0.0,
            )
            cp = pltpu.make_async_copy(
                scw_ref.at[pl.ds(r0, seg), :],
                out_ref.at[pl.ds(r0, seg), pl.ds(0, nh)],
                out_sems.at[0, s],
            )
            cp.start()
            out_cps.append(cp)
            prev_ccw[s].wait()
            sccw_ref[r0:r0 + seg, :] = jnp.maximum(
                stage_ref[r0:r0 + seg, nh:n] + rccw_ref[2, r0:r0 + seg, :], 0.0,
            )
            cp = pltpu.make_async_copy(
                sccw_ref.at[pl.ds(r0, seg), :],
                out_ref.at[pl.ds(r0, seg), pl.ds(nh, nh)],
                out_sems.at[1, s],
            )
            cp.start()
            out_cps.append(cp)
        for cp in out_cps:
            cp.wait()

    return pl.pallas_call(
        body,
        out_shape=jax.ShapeDtypeStruct((m_chunk, n), f32),
        in_specs=[
            pl.BlockSpec(memory_space=pltpu.MemorySpace.HBM),
            pl.BlockSpec(memory_space=pltpu.VMEM),
        ],
        out_specs=pl.BlockSpec(memory_space=pltpu.MemorySpace.HBM),
        scratch_shapes=[
            pltpu.VMEM((m_chunk, nh), f32),
            pltpu.VMEM((m_chunk, nh), f32),
            pltpu.VMEM((N_DEV - 1, m_chunk, nh), f32),
            pltpu.VMEM((N_DEV - 1, m_chunk, nh), f32),
            pltpu.VMEM((3, m_chunk, k_local), f32),
            pltpu.VMEM((m_chunk, n), f32),
            pltpu.SemaphoreType.DMA((SEG,)),
            pltpu.SemaphoreType.DMA((SEG,)),
            pltpu.SemaphoreType.DMA((N_DEV - 1, SEG)),
            pltpu.SemaphoreType.DMA((N_DEV - 1, SEG)),
            pltpu.SemaphoreType.DMA((3,)),
            pltpu.SemaphoreType.DMA((2, SEG)),
        ],
        compiler_params=pltpu.CompilerParams(
            collective_id=0,
            vmem_limit_bytes=128 * 1024 * 1024,
        ),
    )(x, w_mat)


# device time: 154667 ns/iter; 2.0757x vs baseline; 1.0105x over previous
import jax
import jax.numpy as jnp
from jax import lax
from jax.experimental import pallas as pl
from jax.experimental.pallas import tpu as pltpu

N_DEV = 4
SEG = 4


def kernel(x, w_mat):
    m, k_local = x.shape
    _, n = w_mat.shape
    m_chunk = m // N_DEV
    nh = n // 2
    seg = m_chunk // SEG

    f32 = jnp.float32

    def body(x_ref, w_ref, out_ref, scw_ref, sccw_ref, rcw_ref, rccw_ref,
             xch_ref, stage_ref, scw_sems, sccw_sems, rcw_sems, rccw_sems,
             copy_sems, xl_sems, out_sems):
        my = lax.axis_index("i")
        left = (my + N_DEV - 1) % N_DEV
        right = (my + 1) % N_DEV

        barrier_sem = pltpu.get_barrier_semaphore()
        for nbr in [left, right]:
            pl.semaphore_signal(
                barrier_sem, inc=1,
                device_id=(nbr,), device_id_type=pl.DeviceIdType.MESH,
            )
        pl.semaphore_wait(barrier_sem, 2)

        def load_x(c, slot, sem_slot):
            cp = pltpu.make_async_copy(
                x_ref.at[pl.ds(c * m_chunk, m_chunk), :],
                xch_ref.at[slot],
                copy_sems.at[sem_slot],
            )
            cp.start()
            return cp

        def mk_cw(h, s):
            r0 = s * seg
            return pltpu.make_async_remote_copy(
                src_ref=scw_ref.at[pl.ds(r0, seg), :],
                dst_ref=rcw_ref.at[h, pl.ds(r0, seg), :],
                send_sem=scw_sems.at[s],
                recv_sem=rcw_sems.at[h, s],
                device_id=(right,), device_id_type=pl.DeviceIdType.MESH,
            )

        def mk_ccw(h, s):
            r0 = s * seg
            return pltpu.make_async_remote_copy(
                src_ref=sccw_ref.at[pl.ds(r0, seg), :],
                dst_ref=rccw_ref.at[h, pl.ds(r0, seg), :],
                send_sem=sccw_sems.at[s],
                recv_sem=rccw_sems.at[h, s],
                device_id=(left,), device_id_type=pl.DeviceIdType.MESH,
            )

        def load_x_seg(c, slot, s):
            cp = pltpu.make_async_copy(
                x_ref.at[pl.ds(c * m_chunk + s * seg, seg), :],
                xch_ref.at[slot, pl.ds(s * seg, seg), :],
                xl_sems.at[slot, s],
            )
            cp.start()
            return cp

        cps0 = [load_x_seg((my + N_DEV - 1) % N_DEV, 0, s) for s in range(SEG)]
        cps1 = [load_x_seg((my + 1) % N_DEV, 1, s) for s in range(SEG)]
        cp2 = load_x((my + 2) % N_DEV, 2, 2)

        prev_cw = [None] * SEG
        prev_ccw = [None] * SEG
        for s in range(SEG):
            r0 = s * seg
            cps0[s].wait()
            scw_ref[r0:r0 + seg, :] = jnp.dot(
                xch_ref[0, r0:r0 + seg, :], w_ref[:, 0:nh],
                preferred_element_type=f32,
            )
            prev_cw[s] = mk_cw(0, s)
            prev_cw[s].start()
            cps1[s].wait()
            sccw_ref[r0:r0 + seg, :] = jnp.dot(
                xch_ref[1, r0:r0 + seg, :], w_ref[:, nh:n],
                preferred_element_type=f32,
            )
            prev_ccw[s] = mk_ccw(0, s)
            prev_ccw[s].start()

        cp2.wait()
        stage_ref[:, :] = jnp.dot(
            xch_ref[2, :, :], w_ref[:, :], preferred_element_type=f32,
        )

        for s in range(SEG):
            r0 = s * seg
            prev_cw[s].wait()
            scw_ref[r0:r0 + seg, :] = (
                rcw_ref[0, r0:r0 + seg, :] + stage_ref[r0:r0 + seg, 0:nh]
            )
            prev_cw[s] = mk_cw(1, s)
            prev_cw[s].start()
            prev_ccw[s].wait()
            sccw_ref[r0:r0 + seg, :] = (
                rccw_ref[0, r0:r0 + seg, :] + stage_ref[r0:r0 + seg, nh:n]
            )
            prev_ccw[s] = mk_ccw(1, s)
            prev_ccw[s].start()

        cp3 = load_x(my, 2, 2)
        stage_ref[:, 0:nh] = jnp.dot(
            xch_ref[1, :, :], w_ref[:, 0:nh], preferred_element_type=f32,
        )
        stage_ref[:, nh:n] = jnp.dot(
            xch_ref[0, :, :], w_ref[:, nh:n], preferred_element_type=f32,
        )

        for s in range(SEG):
            r0 = s * seg
            prev_cw[s].wait()
            scw_ref[r0:r0 + seg, :] = (
                rcw_ref[1, r0:r0 + seg, :] + stage_ref[r0:r0 + seg, 0:nh]
            )
            prev_cw[s] = mk_cw(2, s)
            prev_cw[s].start()
            prev_ccw[s].wait()
            sccw_ref[r0:r0 + seg, :] = (
                rccw_ref[1, r0:r0 + seg, :] + stage_ref[r0:r0 + seg, nh:n]
            )
            prev_ccw[s] = mk_ccw(2, s)
            prev_ccw[s].start()

        cp3.wait()
        stage_ref[:, :] = jnp.dot(
            xch_ref[2, :, :], w_ref[:, :], preferred_element_type=f32,
        )

        out_cps = []
        for s in range(SEG):
            r0 = s * seg
            prev_cw[s].wait()
            scw_ref[r0:r0 + seg, :] = jnp.maximum(
                stage_ref[r0:r0 + seg, 0:nh] + rcw_ref[2, r0:r0 + seg, :], 0.0,
            )
            cp = pltpu.make_async_copy(
                scw_ref.at[pl.ds(r0, seg), :],
                out_ref.at[pl.ds(r0, seg), pl.ds(0, nh)],
                out_sems.at[0, s],
            )
            cp.start()
            out_cps.append(cp)
            prev_ccw[s].wait()
            sccw_ref[r0:r0 + seg, :] = jnp.maximum(
                stage_ref[r0:r0 + seg, nh:n] + rccw_ref[2, r0:r0 + seg, :], 0.0,
            )
            cp = pltpu.make_async_copy(
                sccw_ref.at[pl.ds(r0, seg), :],
                out_ref.at[pl.ds(r0, seg), pl.ds(nh, nh)],
                out_sems.at[1, s],
            )
            cp.start()
            out_cps.append(cp)
        for cp in out_cps:
            cp.wait()

    return pl.pallas_call(
        body,
        out_shape=jax.ShapeDtypeStruct((m_chunk, n), f32),
        in_specs=[
            pl.BlockSpec(memory_space=pltpu.MemorySpace.HBM),
            pl.BlockSpec(memory_space=pltpu.VMEM),
        ],
        out_specs=pl.BlockSpec(memory_space=pltpu.MemorySpace.HBM),
        scratch_shapes=[
            pltpu.VMEM((m_chunk, nh), f32),
            pltpu.VMEM((m_chunk, nh), f32),
            pltpu.VMEM((N_DEV - 1, m_chunk, nh), f32),
            pltpu.VMEM((N_DEV - 1, m_chunk, nh), f32),
            pltpu.VMEM((3, m_chunk, k_local), f32),
            pltpu.VMEM((m_chunk, n), f32),
            pltpu.SemaphoreType.DMA((SEG,)),
            pltpu.SemaphoreType.DMA((SEG,)),
            pltpu.SemaphoreType.DMA((N_DEV - 1, SEG)),
            pltpu.SemaphoreType.DMA((N_DEV - 1, SEG)),
            pltpu.SemaphoreType.DMA((3,)),
            pltpu.SemaphoreType.DMA((2, SEG)),
            pltpu.SemaphoreType.DMA((2, SEG)),
        ],
        compiler_params=pltpu.CompilerParams(
            collective_id=0,
            vmem_limit_bytes=128 * 1024 * 1024,
        ),
    )(x, w_mat)
